# baseline (speedup 1.0000x reference)
import functools

import jax
import jax.numpy as jnp
from jax import lax
from jax.experimental import pallas as pl
from jax.experimental.pallas import tpu as pltpu
from jax.experimental.pallas import tpu_sc as plsc

F32 = jnp.float32

N_NODES = 10000
N_EDGES = 160000
D = 64
D2 = 128

NC = 2
NS = 16
NW = NC * NS

NP = 10240
EP = 163840
CH = 128
E_CPT = EP // (NW * CH)
E_CPC = EP // (NS * CH)
E_ROWS = EP // CH
NCH = 40
N_CPT = NP // (NW * NCH)
N_ROWS = NP // NCH
PAD_NODE = N_NODES
KF = 5
KFS = 2


def _mesh():
    return plsc.VectorSubcoreMesh(core_axis_name="c", subcore_axis_name="s",
                                  num_cores=NC, num_subcores=NS)


_SC_PARAMS = pltpu.CompilerParams(use_tc_tiling_on_sc=True)


def _pipelined_gather(tbl, idx_v, out, bufs, gsem, wsem, row0, n_chunks, ch):
    n_groups = n_chunks // KF

    def group(g, _):
        gd = []
        for i in range(KF):
            j = g * KF + i
            gd.append(pltpu.async_copy(tbl.at[idx_v.at[j]], bufs.at[i],
                                       gsem.at[i]))
        wd = []
        for i in range(KF):
            j = g * KF + i
            gd[i].wait()
            wd.append(pltpu.async_copy(
                bufs.at[i], out.at[pl.ds((row0 + j) * ch, ch)], wsem))
        for w in wd:
            w.wait()
        return 0

    lax.fori_loop(0, n_groups, group, 0)
    for j in range(n_groups * KF, n_chunks):
        pltpu.sync_copy(tbl.at[idx_v.at[j]], bufs.at[0])
        pltpu.sync_copy(bufs.at[0], out.at[pl.ds((row0 + j) * ch, ch)])


@functools.lru_cache(maxsize=None)
def _make_edge_gather():
    @functools.partial(
        pl.kernel,
        out_type=[jax.ShapeDtypeStruct((EP, D2), F32)] * 2,
        mesh=_mesh(),
        scratch_types=[
            pltpu.VMEM((E_CPT, CH), jnp.int32),
            pltpu.VMEM((E_CPT, CH), jnp.int32),
            pltpu.VMEM((KF, CH, D2), F32),
            pltpu.SemaphoreType.DMA((KF,)),
            pltpu.SemaphoreType.DMA,
        ],
        compiler_params=_SC_PARAMS,
    )
    def edge_gather(tbl, s_hbm, o_hbm, gs_hbm, go_hbm,
                    sidx_v, oidx_v, bufs, gsem, wsem):
        wid = lax.axis_index("c") * NS + lax.axis_index("s")
        row0 = wid * E_CPT
        pltpu.sync_copy(s_hbm.at[pl.ds(row0, E_CPT)], sidx_v)
        pltpu.sync_copy(o_hbm.at[pl.ds(row0, E_CPT)], oidx_v)
        _pipelined_gather(tbl, sidx_v, gs_hbm, bufs, gsem, wsem,
                          row0, E_CPT, CH)
        _pipelined_gather(tbl, oidx_v, go_hbm, bufs, gsem, wsem,
                          row0, E_CPT, CH)

    return edge_gather


@functools.lru_cache(maxsize=None)
def _make_node_gather():
    @functools.partial(
        pl.kernel,
        out_type=jax.ShapeDtypeStruct((NP, D2), F32),
        mesh=_mesh(),
        scratch_types=[
            pltpu.VMEM((N_CPT, NCH), jnp.int32),
            pltpu.VMEM((KF, NCH, D2), F32),
            pltpu.SemaphoreType.DMA((KF,)),
            pltpu.SemaphoreType.DMA,
        ],
        compiler_params=_SC_PARAMS,
    )
    def node_gather(tbl, idx_hbm, out_hbm, idx_v, bufs, gsem, wsem):
        wid = lax.axis_index("c") * NS + lax.axis_index("s")
        row0 = wid * N_CPT
        pltpu.sync_copy(idx_hbm.at[pl.ds(row0, N_CPT)], idx_v)
        _pipelined_gather(tbl, idx_v, out_hbm, bufs, gsem, wsem,
                          row0, N_CPT, NCH)

    return node_gather


@functools.lru_cache(maxsize=None)
def _make_pred_gather():
    @functools.partial(
        pl.kernel,
        out_type=jax.ShapeDtypeStruct((EP, D2), F32),
        mesh=_mesh(),
        scratch_types=[
            pltpu.VMEM((E_CPT, CH), jnp.int32),
            pltpu.VMEM((KF, CH, D2), F32),
            pltpu.SemaphoreType.DMA,
            pltpu.SemaphoreType.DMA,
        ],
        compiler_params=_SC_PARAMS,
    )
    def pred_gather(tblP, p_hbm, out_hbm, idx_v, bufs, gsem, wsem):
        wid = lax.axis_index("c") * NS + lax.axis_index("s")
        row0 = wid * E_CPT
        pltpu.sync_copy(p_hbm.at[pl.ds(row0, E_CPT)], idx_v)
        _pipelined_gather(tblP, idx_v, out_hbm, bufs, gsem, wsem,
                          row0, E_CPT, CH)

    return pred_gather


H = 5000
H_ACC = 5120
DUMP = 5008


@functools.lru_cache(maxsize=None)
def _make_scatter_role():
    npt = H_ACC // NS

    @functools.partial(
        pl.kernel,
        out_type=jax.ShapeDtypeStruct((NC, H_ACC, D2), F32),
        mesh=_mesh(),
        scratch_types=[
            pltpu.VMEM((E_CPC, CH), jnp.int32),
            pltpu.VMEM((KFS, CH, D2), F32),
            pltpu.VMEM_SHARED((H_ACC, D2), F32),
            pltpu.SemaphoreType.DMA((KFS,)),
            pltpu.SemaphoreType.DMA,
        ],
        compiler_params=_SC_PARAMS,
    )
    def scatter_role(zeros_hbm, s_hbm, o_hbm, v_hbm, out_hbm,
                     idx_v, bufs, acc, lsem, ssem):
        cid = lax.axis_index("c")
        sid = lax.axis_index("s")
        pltpu.sync_copy(zeros_hbm.at[pl.ds(sid * npt, npt)],
                        acc.at[pl.ds(sid * npt, npt)])

        row0 = sid * E_CPC

        @pl.when(cid == 0)
        def _():
            pltpu.sync_copy(s_hbm.at[pl.ds(row0, E_CPC)], idx_v)

        @pl.when(cid == 1)
        def _():
            pltpu.sync_copy(o_hbm.at[pl.ds(row0, E_CPC)], idx_v)

        plsc.subcore_barrier()

        n_groups = E_CPC // KFS

        def group(g, _):
            lds = []
            for i in range(KFS):
                j = g * KFS + i
                lds.append(pltpu.async_copy(
                    v_hbm.at[pl.ds((row0 + j) * CH, CH)], bufs.at[i],
                    lsem.at[i]))
            scs = []
            for i in range(KFS):
                j = g * KFS + i
                lds[i].wait()
                scs.append(pltpu.async_copy(
                    bufs.at[i], acc.at[idx_v.at[j]], ssem, add=True))
            for d in scs:
                d.wait()
            return 0

        lax.fori_loop(0, n_groups, group, 0)
        for j in range(E_CPC - E_CPC % KFS, E_CPC):
            pltpu.sync_copy(v_hbm.at[pl.ds((row0 + j) * CH, CH)], bufs.at[0])
            pltpu.sync_copy(bufs.at[0], acc.at[idx_v.at[j]], add=True)
        plsc.subcore_barrier()

        pltpu.sync_copy(acc.at[pl.ds(sid * npt, npt)],
                        out_hbm.at[cid, pl.ds(sid * npt, npt)])

    return scatter_role


def _full(shape):
    return pl.BlockSpec(shape, lambda *_: tuple(0 for _ in shape))


def _prep_body(emb_ref, pemb_ref, w1_ref, b1_ref, t0_ref, tp_ref):
    w1 = w1_ref[...]
    emb = emb_ref[...]
    ta = jnp.dot(emb, w1[0:D], preferred_element_type=F32) + b1_ref[...]
    tb = jnp.dot(emb, w1[2 * D:3 * D], preferred_element_type=F32)
    t0_ref[...] = jnp.concatenate([ta, tb], axis=1)
    tp = jnp.dot(pemb_ref[...], w1[D:2 * D], preferred_element_type=F32)
    tp_ref[...] = jnp.concatenate([tp, jnp.zeros_like(tp)], axis=1)


def _prep_call(emb, pemb, w1, b1):
    return pl.pallas_call(
        _prep_body,
        out_shape=[
            jax.ShapeDtypeStruct((emb.shape[0], D2), F32),
            jax.ShapeDtypeStruct((pemb.shape[0], D2), F32),
        ],
    )(emb, pemb, w1, b1)


_EB = 1024


def _edge_mid(gs_ref, go_ref, p_ref, w2_ref, b2_ref, wpn_ref,
              v_ref, pn_ref, first):
    if first:
        pterm = p_ref[...][:, 0:D]
    else:
        pterm = p_ref[...]
    h = jnp.maximum(gs_ref[...][:, 0:D] + go_ref[...][:, D:D2] + pterm, 0.0)
    t = jnp.dot(h, w2_ref[...], preferred_element_type=F32) + b2_ref[...]
    t = jnp.maximum(t, 0.0)
    v_ref[...] = jnp.concatenate([t[:, 0:D], t[:, 2 * D:3 * D]], axis=1)
    if pn_ref is not None:
        pn_ref[...] = jnp.dot(t[:, D:2 * D], wpn_ref[...],
                              preferred_element_type=F32)


def _edge_body_first(gs_ref, go_ref, p_ref, w2_ref, b2_ref, wpn_ref,
                     v_ref, pn_ref):
    _edge_mid(gs_ref, go_ref, p_ref, w2_ref, b2_ref, wpn_ref,
              v_ref, pn_ref, True)


def _edge_body_mid(gs_ref, go_ref, p_ref, w2_ref, b2_ref, wpn_ref,
                   v_ref, pn_ref):
    _edge_mid(gs_ref, go_ref, p_ref, w2_ref, b2_ref, wpn_ref,
              v_ref, pn_ref, False)


def _edge_body_last(gs_ref, go_ref, p_ref, w2_ref, b2_ref, v_ref):
    _edge_mid(gs_ref, go_ref, p_ref, w2_ref, b2_ref, None, v_ref, None, False)


def _edge_call(gs, go, p, w2, b2, wpn, first):
    grid = (EP // _EB,)
    eb = pl.BlockSpec((_EB, D2), lambda i: (i, 0))
    pb_full = pl.BlockSpec((_EB, D2), lambda i: (i, 0))
    pb_plain = pl.BlockSpec((_EB, D), lambda i: (i, 0))
    pnb = pl.BlockSpec((_EB, D), lambda i: (i, 0))
    pspec = pb_full if first else pb_plain
    if wpn is None:
        return pl.pallas_call(
            _edge_body_last,
            grid=grid,
            in_specs=[eb, eb, pspec, _full((D, 3 * D)), _full((1, 3 * D))],
            out_specs=[eb],
            out_shape=[jax.ShapeDtypeStruct((EP, D2), F32)],
        )(gs, go, p, w2, b2)[0]
    body = _edge_body_first if first else _edge_body_mid
    return pl.pallas_call(
        body,
        grid=grid,
        in_specs=[eb, eb, pspec, _full((D, 3 * D)), _full((1, 3 * D)),
                  _full((D, D))],
        out_specs=[eb, pnb],
        out_shape=[jax.ShapeDtypeStruct((EP, D2), F32),
                   jax.ShapeDtypeStruct((EP, D), F32)],
    )(gs, go, p, w2, b2, wpn)


def _inv_body(clo_ref, chi_ref, inv_ref):
    cl = clo_ref[0][0:H, 0:D] + clo_ref[1][0:H, D:D2]
    ch_ = chi_ref[0][:, 0:D] + chi_ref[1][:, D:D2]
    inv_ref[...] = jnp.maximum(jnp.concatenate([cl, ch_], axis=0), 1.0)


def _inv_call(clo, chi):
    return pl.pallas_call(
        _inv_body,
        out_shape=jax.ShapeDtypeStruct((H + H_ACC, D), F32),
    )(clo, chi)


def _pool_mlp(plo_ref, phi_ref, inv_ref,
              w2a_ref, b2a_ref, w2b_ref, b2b_ref):
    lo = plo_ref[0][0:H, 0:D] + plo_ref[1][0:H, D:D2]
    hi = phi_ref[0][:, 0:D] + phi_ref[1][:, D:D2]
    pooled = jnp.concatenate([lo, hi], axis=0)
    x = pooled / inv_ref[...]
    x = jnp.maximum(jnp.dot(x, w2a_ref[...], preferred_element_type=F32)
                    + b2a_ref[...], 0.0)
    return jnp.maximum(jnp.dot(x, w2b_ref[...], preferred_element_type=F32)
                       + b2b_ref[...], 0.0)


def _node_body(plo_ref, phi_ref, inv_ref, w2a_ref, b2a_ref,
               w2b_ref, b2b_ref, wsn_ref, b1n_ref, won_ref, t_ref):
    obj = _pool_mlp(plo_ref, phi_ref, inv_ref,
                    w2a_ref, b2a_ref, w2b_ref, b2b_ref)
    ta = jnp.dot(obj, wsn_ref[...], preferred_element_type=F32) + b1n_ref[...]
    tb = jnp.dot(obj, won_ref[...], preferred_element_type=F32)
    t = jnp.concatenate([ta, tb], axis=1)
    t_ref[...] = jnp.concatenate(
        [t, jnp.zeros((NP - H - H_ACC, D2), F32)], axis=0)


def _node_call(plo, phi, inv, w2a, b2a, w2b, b2b, wsn, b1n, won):
    return pl.pallas_call(
        _node_body,
        out_shape=jax.ShapeDtypeStruct((NP, D2), F32),
    )(plo, phi, inv, w2a, b2a, w2b, b2b, wsn, b1n, won)


def _node_last_body(plo_ref, phi_ref, inv_ref, w2a_ref, b2a_ref,
                    w2b_ref, b2b_ref, wb1_ref, bb1_ref, wb2_ref, bb2_ref,
                    out_ref):
    obj = _pool_mlp(plo_ref, phi_ref, inv_ref,
                    w2a_ref, b2a_ref, w2b_ref, b2b_ref)
    y = jnp.maximum(jnp.dot(obj, wb1_ref[...], preferred_element_type=F32)
                    + bb1_ref[...], 0.0)
    b = jnp.maximum(
        jnp.dot(y, wb2_ref[...], preferred_element_type=F32) + bb2_ref[...], 0.0)
    out_ref[...] = jnp.concatenate(
        [b, jnp.zeros((NP - H - H_ACC, D), F32)], axis=0)


def _node_last_call(plo, phi, inv, w2a, b2a, w2b, b2b, wb1, bb1, wb2, bb2):
    return pl.pallas_call(
        _node_last_body,
        out_shape=jax.ShapeDtypeStruct((NP, D), F32),
    )(plo, phi, inv, w2a, b2a, w2b, b2b, wb1, bb1, wb2, bb2)


def _dep(arr, token):
    out, _ = lax.optimization_barrier((arr, token))
    return out


def kernel(params, objs, triples):
    s = triples[:, 0]
    p = triples[:, 1]
    o = triples[:, 2]
    pad_e = EP - N_EDGES
    s2 = jnp.concatenate([s, jnp.full((pad_e,), PAD_NODE, jnp.int32)]
                         ).reshape(E_ROWS, CH)
    o2 = jnp.concatenate([o, jnp.full((pad_e,), PAD_NODE, jnp.int32)]
                         ).reshape(E_ROWS, CH)
    p2 = jnp.concatenate([p, jnp.zeros((pad_e,), jnp.int32)]
                         ).reshape(E_ROWS, CH)
    objs2 = jnp.concatenate([objs, jnp.zeros((NP - N_NODES,), jnp.int32)]
                            ).reshape(N_ROWS, NCH)

    obj_emb = jnp.pad(params["obj_emb"], ((0, 3), (0, 0)))
    pred_emb = jnp.pad(params["pred_emb"], ((0, 2), (0, 0)))

    gconv = params["gconv"]
    w1 = [g["net1"][0] for g in gconv]
    b1 = [g["net1"][1].reshape(1, D) for g in gconv]
    w2 = [g["net1"][2] for g in gconv]
    b2 = [g["net1"][3].reshape(1, 3 * D) for g in gconv]
    w2a = [g["net2"][0] for g in gconv]
    b2a = [g["net2"][1].reshape(1, D) for g in gconv]
    w2b = [g["net2"][2] for g in gconv]
    b2b = [g["net2"][3].reshape(1, D) for g in gconv]

    box = params["box_net"]
    wb1, bb1 = box[0], box[1].reshape(1, D)
    wb2 = jnp.pad(box[2], ((0, 0), (0, D - 4)))
    bb2 = jnp.pad(box[3], ((0, D - 4),)).reshape(1, D)

    t0, tabP = _prep_call(obj_emb, pred_emb, w1[0], b1[0])

    node_gather = _make_node_gather()
    edge_gather = _make_edge_gather()
    scatter_role = _make_scatter_role()

    T = node_gather(t0, objs2)
    tabP_big = jnp.pad(tabP, ((0, NP - tabP.shape[0]), (0, 0)))
    P, _unused = edge_gather(tabP_big, _dep(p2, T[0, 0]), p2)

    dump = jnp.int32(DUMP)
    s_lo = jnp.where(s2 < H, s2, dump)
    s_hi = jnp.where(s2 >= H, s2 - H, dump)
    o_lo = jnp.where(o2 < H, o2, dump)
    o_hi = jnp.where(o2 >= H, o2 - H, dump)

    ones_ep = jnp.ones((EP, D2), F32)
    zeros_acc = jnp.zeros((H_ACC, D2), F32)
    counts_lo = scatter_role(zeros_acc, _dep(s_lo, P[0, 0]), o_lo, ones_ep)
    counts_hi = scatter_role(zeros_acc, _dep(s_hi, counts_lo[0, 0, 0]),
                             o_hi, ones_ep)
    inv = _inv_call(counts_lo, counts_hi)
    tok = counts_hi[0, 0, 0]

    for li in range(5):
        gs, go = edge_gather(T, _dep(s2, tok), o2)
        if li < 4:
            v, P = _edge_call(gs, go, P, w2[li], b2[li],
                              w1[li + 1][D:2 * D], li == 0)
        else:
            v = _edge_call(gs, go, P, w2[li], b2[li], None, False)
        p_lo = scatter_role(zeros_acc, _dep(s_lo, gs[0, 0]), o_lo, v)
        p_hi = scatter_role(zeros_acc, _dep(s_hi, p_lo[0, 0, 0]), o_hi, v)
        tok = p_hi[0, 0, 0]
        if li < 4:
            T = _node_call(p_lo, p_hi, inv,
                           w2a[li], b2a[li], w2b[li], b2b[li],
                           w1[li + 1][0:D], b1[li + 1],
                           w1[li + 1][2 * D:3 * D])
        else:
            boxes = _node_last_call(p_lo, p_hi, inv,
                                    w2a[li], b2a[li], w2b[li], b2b[li],
                                    wb1, bb1, wb2, bb2)

    return boxes[:N_NODES, :4]
